# Initial kernel scaffold; baseline (speedup 1.0000x reference)
import jax
import jax.numpy as jnp
from jax.experimental import pallas as pl


def kernel(x, Wq, Wk, Wv, Wout, W_nudge, chamber_bonus, simple_roots):
    raise NotImplementedError("write your pallas kernel here")


# speedup vs baseline: 1.0444x; 1.0444x over previous
import math

import jax
import jax.numpy as jnp
import numpy as np
from jax.experimental import pallas as pl

H = 12
DH = 4
DV = 16
SCALE = 1.0 / math.sqrt(DH)
BQ = 256


def _np_group_sum():
    g = np.zeros((H * DH, H * DH), np.float32)
    for h in range(H):
        g[h * DH:(h + 1) * DH, h * DH:(h + 1) * DH] = 1.0
    return g


def _np_expand():
    e = np.zeros((H * DH, H * 8), np.float32)
    for h in range(H):
        for d in range(DH):
            e[h * DH + d, h * 8 + d] = 1.0
    return e


def _np_ones_col():
    o = np.zeros((1, H * 8), np.float32)
    for h in range(H):
        o[0, h * 8 + DH] = 1.0
    return o


_G = _np_group_sum()
_E = _np_expand()
_ONEC = _np_ones_col()


def _prep_kernel(x_ref, wqt_ref, wkt_ref, wvt_ref, bdn_ref, g_ref, e_ref,
                 onec_ref, qa_ref, ka_ref, v_ref):
    x = x_ref[...]
    g = g_ref[...]

    def norm(a):
        n2 = jnp.dot(a * a, g, preferred_element_type=jnp.float32)
        return a / jnp.maximum(jnp.sqrt(n2), 1e-12)

    q = jnp.dot(x, wqt_ref[...], preferred_element_type=jnp.float32)
    k = jnp.dot(x, wkt_ref[...], preferred_element_type=jnp.float32)
    v_ref[...] = jnp.dot(x, wvt_ref[...], preferred_element_type=jnp.float32)
    qn = norm(jnp.dot(norm(q), bdn_ref[...],
                      preferred_element_type=jnp.float32))
    e = e_ref[...]
    qa_ref[...] = jnp.dot(qn * SCALE, e,
                          preferred_element_type=jnp.float32) + onec_ref[...]
    ka_ref[...] = jnp.dot(norm(k), e, preferred_element_type=jnp.float32)


def _attn_kernel(qa_ref, kat_ref, v_ref, o_ref):
    qb = pl.program_id(1)
    t = kat_ref.shape[2]
    s = jnp.dot(qa_ref[0], kat_ref[0],
                preferred_element_type=jnp.float32)
    rows = qb * BQ + jax.lax.broadcasted_iota(jnp.int32, (BQ, t), 0)
    cols = jax.lax.broadcasted_iota(jnp.int32, (BQ, t), 1)
    s = jnp.where(cols > rows, jnp.float32(-1e30), s)
    m = jnp.max(s, axis=1, keepdims=True)
    p = jnp.exp(s - m)
    l = jnp.sum(p, axis=1, keepdims=True)
    o_ref[0] = jnp.dot(p, v_ref[0],
                       preferred_element_type=jnp.float32) / l


def _proj_kernel(o_ref, wot_ref, y_ref):
    y_ref[...] = jnp.dot(o_ref[...], wot_ref[...],
                         preferred_element_type=jnp.float32)


def kernel(x, Wq, Wk, Wv, Wout, W_nudge, chamber_bonus, simple_roots):
    b, t, d_model = x.shape
    x2 = x.reshape(t, d_model)

    bdn = (jnp.zeros((H, DH, H, DH), jnp.float32)
           .at[jnp.arange(H), :, jnp.arange(H), :].set(W_nudge)
           .reshape(H * DH, H * DH))

    qa, ka, v = pl.pallas_call(
        _prep_kernel,
        out_shape=[
            jax.ShapeDtypeStruct((t, H * 8), jnp.float32),
            jax.ShapeDtypeStruct((t, H * 8), jnp.float32),
            jax.ShapeDtypeStruct((t, H * DV), jnp.float32),
        ],
    )(x2, Wq.T, Wk.T, Wv.T, bdn, jnp.asarray(_G), jnp.asarray(_E),
      jnp.asarray(_ONEC))

    ka3 = ka.reshape(t, H, 8)
    kn = ka3[..., :DH]
    kd = jnp.einsum('thd,rd->thr', kn, simple_roots)
    ss = jax.nn.sigmoid(kd * 3.0)
    cw = jnp.ones((t, H, 16), jnp.float32)
    for bit in range(4):
        sb = ss[..., bit][..., None]
        mask = ((jnp.arange(16) >> bit) & 1).astype(jnp.float32)
        cw = cw * (sb * mask + (1 - sb) * (1 - mask))
    bonus = (cw * chamber_bonus[None]).sum(-1)
    ka3 = ka3 + bonus[..., None] * jnp.eye(8, dtype=jnp.float32)[DH]

    qa3 = qa.reshape(t, H, 8).transpose(1, 0, 2)
    kat3 = ka3.transpose(1, 2, 0)
    v3 = v.reshape(t, H, DV).transpose(1, 0, 2)

    nq = t // BQ
    o3 = pl.pallas_call(
        _attn_kernel,
        grid=(H, nq),
        in_specs=[
            pl.BlockSpec((1, BQ, 8), lambda h, qb: (h, qb, 0)),
            pl.BlockSpec((1, 8, t), lambda h, qb: (h, 0, 0)),
            pl.BlockSpec((1, t, DV), lambda h, qb: (h, 0, 0)),
        ],
        out_specs=pl.BlockSpec((1, BQ, DV), lambda h, qb: (h, qb, 0)),
        out_shape=jax.ShapeDtypeStruct((H, t, DV), jnp.float32),
    )(qa3, kat3, v3)

    o2 = o3.transpose(1, 0, 2).reshape(t, H * DV)
    y = pl.pallas_call(
        _proj_kernel,
        out_shape=jax.ShapeDtypeStruct((t, d_model), jnp.float32),
    )(o2, Wout.T)
    return y.reshape(b, t, d_model)


# speedup vs baseline: 2.2649x; 2.1687x over previous
import math

import jax
import jax.numpy as jnp
import numpy as np
from jax.experimental import pallas as pl

H = 12
DH = 4
DV = 16
SCALE = 1.0 / math.sqrt(DH)
LOG2E = 1.4426950408889634
BQ = 256


def _np_group_sum():
    g = np.zeros((H * DH, H * DH), np.float32)
    for h in range(H):
        g[h * DH:(h + 1) * DH, h * DH:(h + 1) * DH] = 1.0
    return g


def _np_expand():
    e = np.zeros((H * DH, H * 8), np.float32)
    for h in range(H):
        for d in range(DH):
            e[h * DH + d, h * 8 + d] = 1.0
    return e


def _np_ones_col():
    o = np.zeros((1, H * 8), np.float32)
    for h in range(H):
        o[0, h * 8 + DH] = LOG2E
    return o


_G = _np_group_sum()
_E = _np_expand()
_ONEC = _np_ones_col()


def _prep_kernel(x_ref, wqt_ref, wkt_ref, wvt_ref, bdn_ref, g_ref, e_ref,
                 onec_ref, qa_ref, ka_ref, v_ref):
    x = x_ref[...]
    g = g_ref[...]

    def norm(a):
        n2 = jnp.dot(a * a, g, preferred_element_type=jnp.float32)
        return a / jnp.maximum(jnp.sqrt(n2), 1e-12)

    q = jnp.dot(x, wqt_ref[...], preferred_element_type=jnp.float32)
    k = jnp.dot(x, wkt_ref[...], preferred_element_type=jnp.float32)
    v_ref[...] = jnp.dot(x, wvt_ref[...], preferred_element_type=jnp.float32)
    qn = norm(jnp.dot(norm(q), bdn_ref[...],
                      preferred_element_type=jnp.float32))
    e = e_ref[...]
    qa_ref[...] = jnp.dot(qn * (SCALE * LOG2E), e,
                          preferred_element_type=jnp.float32) + onec_ref[...]
    ka_ref[...] = jnp.dot(norm(k), e, preferred_element_type=jnp.float32)


def _attn_kernel(qa_ref, kat_ref, v_ref, o_ref):
    kat = kat_ref[0]
    v = v_ref[0]
    t = kat.shape[1]
    r = jax.lax.broadcasted_iota(jnp.int32, (BQ, BQ), 0)
    c = jax.lax.broadcasted_iota(jnp.int32, (BQ, BQ), 1)
    amask = jnp.where(c > r, jnp.float32(-1e30), jnp.float32(0.0))
    for qb in range(t // BQ):
        lo = qb * BQ
        qa = qa_ref[0, lo:lo + BQ, :]
        s1 = jnp.dot(qa, kat[:, lo:lo + BQ],
                     preferred_element_type=jnp.float32) + amask
        p1 = jnp.exp2(s1)
        oa = jnp.dot(p1, v[lo:lo + BQ, :],
                     preferred_element_type=jnp.float32)
        if qb > 0:
            s0 = jnp.dot(qa, kat[:, :lo],
                         preferred_element_type=jnp.float32)
            p0 = jnp.exp2(s0)
            oa = oa + jnp.dot(p0, v[:lo, :],
                              preferred_element_type=jnp.float32)
        o_ref[0, lo:lo + BQ, :] = oa[:, :DV] / oa[:, DV:DV + 1]


def _proj_kernel(o_ref, wot_ref, y_ref):
    y_ref[...] = jnp.dot(o_ref[...], wot_ref[...],
                         preferred_element_type=jnp.float32)


def kernel(x, Wq, Wk, Wv, Wout, W_nudge, chamber_bonus, simple_roots):
    b, t, d_model = x.shape
    x2 = x.reshape(t, d_model)

    bdn = (jnp.zeros((H, DH, H, DH), jnp.float32)
           .at[jnp.arange(H), :, jnp.arange(H), :].set(W_nudge)
           .reshape(H * DH, H * DH))

    qa, ka, v = pl.pallas_call(
        _prep_kernel,
        out_shape=[
            jax.ShapeDtypeStruct((t, H * 8), jnp.float32),
            jax.ShapeDtypeStruct((t, H * 8), jnp.float32),
            jax.ShapeDtypeStruct((t, H * DV), jnp.float32),
        ],
    )(x2, Wq.T, Wk.T, Wv.T, bdn, jnp.asarray(_G), jnp.asarray(_E),
      jnp.asarray(_ONEC))

    ka3 = ka.reshape(t, H, 8)
    kn = ka3[..., :DH]
    kd = jnp.einsum('thd,rd->thr', kn, simple_roots)
    ss = jax.nn.sigmoid(kd * 3.0)
    cw = jnp.ones((t, H, 16), jnp.float32)
    for bit in range(4):
        sb = ss[..., bit][..., None]
        mask = ((jnp.arange(16) >> bit) & 1).astype(jnp.float32)
        cw = cw * (sb * mask + (1 - sb) * (1 - mask))
    bonus = (cw * chamber_bonus[None]).sum(-1)
    ka3 = ka3 + bonus[..., None] * jnp.eye(8, dtype=jnp.float32)[DH]

    qa3 = qa.reshape(t, H, 8).transpose(1, 0, 2).astype(jnp.bfloat16)
    kat3 = ka3.transpose(1, 2, 0).astype(jnp.bfloat16)
    v3 = v.reshape(t, H, DV).transpose(1, 0, 2)
    v3 = jnp.concatenate(
        [v3, jnp.ones((H, t, 1), jnp.float32),
         jnp.zeros((H, t, 32 - DV - 1), jnp.float32)], axis=-1)

    o3 = pl.pallas_call(
        _attn_kernel,
        grid=(H,),
        in_specs=[
            pl.BlockSpec((1, t, 8), lambda h: (h, 0, 0)),
            pl.BlockSpec((1, 8, t), lambda h: (h, 0, 0)),
            pl.BlockSpec((1, t, 32), lambda h: (h, 0, 0)),
        ],
        out_specs=pl.BlockSpec((1, t, DV), lambda h: (h, 0, 0)),
        out_shape=jax.ShapeDtypeStruct((H, t, DV), jnp.float32),
    )(qa3, kat3, v3)

    o2 = o3.transpose(1, 0, 2).reshape(t, H * DV)
    y = pl.pallas_call(
        _proj_kernel,
        out_shape=jax.ShapeDtypeStruct((t, d_model), jnp.float32),
    )(o2, Wout.T)
    return y.reshape(b, t, d_model)
